# baseline (speedup 1.0000x reference)
import functools

import jax
import jax.numpy as jnp
from jax.experimental import pallas as pl
from jax.experimental.pallas import tpu as pltpu

N_NODES = 50000
N_EDGES = 800000
N_REL = 3
N_GRAPHS = 64
HID = 64
NODE_BLK = 2000
EDGE_BLK = 4000
N_NODE_BLKS = N_NODES // NODE_BLK
N_EDGE_BLKS = N_EDGES // EDGE_BLK
PACKED_ROWS = N_NODES // 2


def _dense1_body(x_ref, se_ref, ce_ref, w0_ref, b0_ref, wroot_ref, b1_ref,
                 wrel_ref, root_ref, hw_ref):
    iota8 = jax.lax.broadcasted_iota(jnp.int32, (1, 8), 1)
    ohs = (x_ref[:, 0:1] == iota8).astype(jnp.float32)
    ohc = (x_ref[:, 1:2] == iota8).astype(jnp.float32)
    a = jnp.dot(se_ref[...], w0_ref[0:8, :], preferred_element_type=jnp.float32)
    b = jnp.dot(ce_ref[...], w0_ref[8:16, :], preferred_element_type=jnp.float32)
    h = jnp.dot(ohs, a, preferred_element_type=jnp.float32)
    h = h + jnp.dot(ohc, b, preferred_element_type=jnp.float32)
    h = jnp.maximum(h + b0_ref[...], 0.0)
    root_ref[...] = jnp.dot(h, wroot_ref[...],
                            preferred_element_type=jnp.float32) + b1_ref[...]
    for r in range(N_REL):
        hw_ref[r] = jnp.dot(h, wrel_ref[r], preferred_element_type=jnp.float32)


def _dense2_body(root_ref, acc_ref, wroot_ref, b_ref, wrel_ref,
                 root_out_ref, hw_ref):
    h = jnp.maximum(root_ref[...] + acc_ref[...], 0.0)
    root_out_ref[...] = jnp.dot(h, wroot_ref[...],
                                preferred_element_type=jnp.float32) + b_ref[...]
    for r in range(N_REL):
        hw_ref[r] = jnp.dot(h, wrel_ref[r], preferred_element_type=jnp.float32)


def _count_body(dst_ref, tp_ref, cnt_ref):
    @pl.when(pl.program_id(0) == 0)
    def _():
        cnt_ref[...] = jnp.zeros_like(cnt_ref)

    iota128 = jax.lax.broadcasted_iota(jnp.int32, (1, 128), 1)

    def body(e, carry):
        d = dst_ref[0, 0, e]
        t = tp_ref[0, 0, e]
        lane = t + 64 * jax.lax.rem(d, 2)
        row = cnt_ref[pl.ds(jax.lax.div(d, 2), 1), :]
        cnt_ref[pl.ds(jax.lax.div(d, 2), 1), :] = (
            row + (iota128 == lane).astype(jnp.float32))
        return carry

    jax.lax.fori_loop(0, EDGE_BLK, body, 0)


def _edge_body(src_ref, dst_ref, tp_ref, hw_ref, cnt_ref, acc_ref):
    r = pl.program_id(0)

    @pl.when((r == 0) & (pl.program_id(1) == 0))
    def _():
        acc_ref[...] = jnp.zeros_like(acc_ref)

    iota128 = jax.lax.broadcasted_iota(jnp.int32, (1, 128), 1)
    zeros64 = jnp.zeros((1, HID), jnp.float32)

    def body(e, carry):
        t = tp_ref[0, 0, e]

        @pl.when(t == r)
        def _():
            s = src_ref[0, 0, e]
            d = dst_ref[0, 0, e]
            hw_row = hw_ref[0, pl.ds(jax.lax.div(s, 2), 1), :]
            msg = jnp.where(jax.lax.rem(s, 2) == 0,
                            hw_row[:, :HID], hw_row[:, HID:])
            crow = cnt_ref[pl.ds(jax.lax.div(d, 2), 1), :]
            lane = t + 64 * jax.lax.rem(d, 2)
            c = jnp.sum(crow * (iota128 == lane).astype(jnp.float32),
                        axis=1, keepdims=True)
            contrib = msg * (1.0 / jnp.maximum(c, 1.0))
            add = jnp.where(jax.lax.rem(d, 2) == 0,
                            jnp.concatenate([contrib, zeros64], axis=1),
                            jnp.concatenate([zeros64, contrib], axis=1))
            arow = acc_ref[pl.ds(jax.lax.div(d, 2), 1), :]
            acc_ref[pl.ds(jax.lax.div(d, 2), 1), :] = arow + add

        return carry

    jax.lax.fori_loop(0, EDGE_BLK, body, 0)


def _pool_body(root_ref, acc_ref, batch_ref, wc_ref, bc_ref,
               out_ref, sums_ref, cnts_ref):
    @pl.when(pl.program_id(0) == 0)
    def _():
        sums_ref[...] = jnp.zeros_like(sums_ref)
        cnts_ref[...] = jnp.zeros_like(cnts_ref)

    h = jnp.maximum(root_ref[...] + acc_ref[...], 0.0)
    iota64 = jax.lax.broadcasted_iota(jnp.int32, (1, N_GRAPHS), 1)
    oh = (batch_ref[...] == iota64).astype(jnp.float32)
    dn = (((0,), (0,)), ((), ()))
    sums_ref[...] += jax.lax.dot_general(oh, h, dn,
                                         preferred_element_type=jnp.float32)
    ones = jnp.ones((NODE_BLK, 1), jnp.float32)
    cnts_ref[...] += jax.lax.dot_general(oh, ones, dn,
                                         preferred_element_type=jnp.float32)

    @pl.when(pl.program_id(0) == N_NODE_BLKS - 1)
    def _():
        pooled = sums_ref[...] / jnp.maximum(cnts_ref[...], 1.0)
        out_ref[...] = jnp.dot(pooled, wc_ref[...],
                               preferred_element_type=jnp.float32) + bc_ref[...]


def _pack(hw):
    return hw.reshape(N_REL, PACKED_ROWS, 2 * HID)


def _unpack(acc):
    return acc.reshape(N_NODES, HID)


_F32 = jnp.float32


def _edge_call(hw_packed, cnt, srcs, dsts, tps):
    smem = lambda: pl.BlockSpec((1, 1, EDGE_BLK), lambda r, i: (i, 0, 0),
                                memory_space=pltpu.SMEM)
    full = lambda shape: pl.BlockSpec(shape, lambda r, i: tuple(0 for _ in shape))
    return pl.pallas_call(
        _edge_body,
        grid=(N_REL, N_EDGE_BLKS),
        in_specs=[smem(), smem(), smem(),
                  pl.BlockSpec((1, PACKED_ROWS, 2 * HID),
                               lambda r, i: (r, 0, 0)),
                  full((PACKED_ROWS, 2 * HID))],
        out_specs=full((PACKED_ROWS, 2 * HID)),
        out_shape=jax.ShapeDtypeStruct((PACKED_ROWS, 2 * HID), _F32),
    )(srcs, dsts, tps, hw_packed, cnt)


def kernel(x, edge_index, edge_type, batch, shape_emb, color_emb, W0, b0,
           Wrel1, Wroot1, b1, Wrel2, Wroot2, b2, Wc, bc):
    srcs = edge_index[0].reshape(N_EDGE_BLKS, 1, EDGE_BLK)
    dsts = edge_index[1].reshape(N_EDGE_BLKS, 1, EDGE_BLK)
    tps = edge_type.reshape(N_EDGE_BLKS, 1, EDGE_BLK)
    b0r, b1r, b2r = b0.reshape(1, HID), b1.reshape(1, HID), b2.reshape(1, HID)
    bcr = bc.reshape(1, -1)

    blk = lambda: pl.BlockSpec((NODE_BLK, HID), lambda i: (i, 0))
    full = lambda shape: pl.BlockSpec(shape, lambda i: tuple(0 for _ in shape))
    hw_spec = pl.BlockSpec((N_REL, NODE_BLK, HID), lambda i: (0, i, 0))
    smem = lambda: pl.BlockSpec((1, 1, EDGE_BLK), lambda i: (i, 0, 0),
                                memory_space=pltpu.SMEM)

    root1, hw1 = pl.pallas_call(
        _dense1_body,
        grid=(N_NODE_BLKS,),
        in_specs=[pl.BlockSpec((NODE_BLK, 2), lambda i: (i, 0)),
                  full((8, 8)), full((8, 8)), full((16, HID)),
                  full((1, HID)), full((HID, HID)), full((1, HID)),
                  full((N_REL, HID, HID))],
        out_specs=[blk(), hw_spec],
        out_shape=[jax.ShapeDtypeStruct((N_NODES, HID), _F32),
                   jax.ShapeDtypeStruct((N_REL, N_NODES, HID), _F32)],
    )(x, shape_emb, color_emb, W0, b0r, Wroot1, b1r, Wrel1)

    cnt = pl.pallas_call(
        _count_body,
        grid=(N_EDGE_BLKS,),
        in_specs=[smem(), smem()],
        out_specs=full((PACKED_ROWS, 2 * HID)),
        out_shape=jax.ShapeDtypeStruct((PACKED_ROWS, 2 * HID), _F32),
    )(dsts, tps)

    acc1 = _edge_call(_pack(hw1), cnt, srcs, dsts, tps)

    root2, hw2 = pl.pallas_call(
        _dense2_body,
        grid=(N_NODE_BLKS,),
        in_specs=[blk(), blk(), full((HID, HID)), full((1, HID)),
                  full((N_REL, HID, HID))],
        out_specs=[blk(), hw_spec],
        out_shape=[jax.ShapeDtypeStruct((N_NODES, HID), _F32),
                   jax.ShapeDtypeStruct((N_REL, N_NODES, HID), _F32)],
    )(root1, _unpack(acc1), Wroot2, b2r, Wrel2)

    acc2 = _edge_call(_pack(hw2), cnt, srcs, dsts, tps)

    out, _, _ = pl.pallas_call(
        _pool_body,
        grid=(N_NODE_BLKS,),
        in_specs=[blk(), blk(),
                  pl.BlockSpec((NODE_BLK, 1), lambda i: (i, 0)),
                  full((HID, 10)), full((1, 10))],
        out_specs=[full((N_GRAPHS, 10)), full((N_GRAPHS, HID)),
                   full((N_GRAPHS, 1))],
        out_shape=[jax.ShapeDtypeStruct((N_GRAPHS, 10), _F32),
                   jax.ShapeDtypeStruct((N_GRAPHS, HID), _F32),
                   jax.ShapeDtypeStruct((N_GRAPHS, 1), _F32)],
    )(root2, _unpack(acc2), batch.reshape(N_NODES, 1), Wc, bcr)
    return out
